# baseline (speedup 1.0000x reference)
import functools

import jax
import jax.numpy as jnp
from jax import lax
from jax.experimental import pallas as pl
from jax.experimental.pallas import tpu as pltpu
from jax.experimental.pallas import tpu_sc as plsc

_NC = 2
_NS = 16
_NW = _NC * _NS


def _build_gather(B: int, V: int, E: int, C: int):
    assert B % (_NW * C) == 0
    b_per_w = B // _NW
    nchunks = b_per_w // C
    assert nchunks >= 4 and nchunks % 2 == 0

    mesh = plsc.VectorSubcoreMesh(core_axis_name="c", subcore_axis_name="s")

    @functools.partial(
        pl.kernel,
        out_type=jax.ShapeDtypeStruct((B, E), jnp.float32),
        mesh=mesh,
        compiler_params=pltpu.CompilerParams(use_tc_tiling_on_sc=False),
        scratch_types=[
            pltpu.VMEM((C,), jnp.int32),
            pltpu.VMEM((C,), jnp.int32),
            pltpu.VMEM((C, E), jnp.float32),
            pltpu.VMEM((C, E), jnp.float32),
            pltpu.SemaphoreType.DMA,
            pltpu.SemaphoreType.DMA,
            pltpu.SemaphoreType.DMA,
            pltpu.SemaphoreType.DMA,
            pltpu.SemaphoreType.DMA,
            pltpu.SemaphoreType.DMA,
        ],
    )
    def gather(idx_hbm, table_hbm, out_hbm, idx_v0, idx_v1, rows_v0, rows_v1, *sems):
        idx_v = (idx_v0, idx_v1)
        rows_v = (rows_v0, rows_v1)
        sem_i = sems[0:2]
        sem_g = sems[2:4]
        sem_o = sems[4:6]
        wid = lax.axis_index("s") * _NC + lax.axis_index("c")
        base = wid * b_per_w

        def start_idx(g, b):
            pltpu.make_async_copy(
                idx_hbm.at[pl.ds(base + g * C, C)], idx_v[b], sem_i[b]
            ).start()

        def wait_idx(b):
            pltpu.make_async_copy(
                idx_hbm.at[pl.ds(base, C)], idx_v[b], sem_i[b]
            ).wait()

        def start_gather(b):
            pltpu.make_async_copy(
                table_hbm.at[idx_v[b]], rows_v[b], sem_g[b]
            ).start()

        def wait_gather(b):
            pltpu.make_async_copy(
                table_hbm.at[idx_v[b]], rows_v[b], sem_g[b]
            ).wait()

        def start_out(g, b):
            pltpu.make_async_copy(
                rows_v[b], out_hbm.at[pl.ds(base + g * C, C)], sem_o[b]
            ).start()

        def wait_out(b):
            pltpu.make_async_copy(
                rows_v[b], out_hbm.at[pl.ds(base, C)], sem_o[b]
            ).wait()

        def step(g, b, first=False, last=False):
            nb = 1 - b
            if not first:
                wait_out(nb)
            wait_idx(nb)
            start_gather(nb)
            wait_gather(b)
            if not last:
                start_idx(g + 2, b)
            start_out(g, b)

        start_idx(0, 0)
        start_idx(1, 1)
        wait_idx(0)
        start_gather(0)
        step(0, 0, first=True)
        step(1, 1)

        def loop_body(i, _):
            g = 2 * i
            step(g, 0)
            step(g + 1, 1)
            return _

        lax.fori_loop(1, nchunks // 2 - 1, loop_body, 0, unroll=False)

        step(nchunks - 2, 0, last=True)
        wait_gather(1)
        start_out(nchunks - 1, 1)
        wait_out(0)
        wait_out(1)

    return gather


def _tc_relayout(N: int, W: int):
    rows_per_blk = W // 128 * 8

    def body(x_ref, o_ref):
        o_ref[...] = x_ref[...].reshape(8, W)

    return pl.pallas_call(
        body,
        grid=(N // 8,),
        in_specs=[pl.BlockSpec((rows_per_blk, 128), lambda i: (i, 0))],
        out_specs=pl.BlockSpec((8, W), lambda i: (i, 0)),
        out_shape=jax.ShapeDtypeStruct((N, W), jnp.float32),
    )


def kernel(p, table):
    N, D = p.shape
    V, E = table.shape
    flat = _build_gather(N * D, V, E, C=800)(p.reshape(N * D), table)
    x = flat.reshape(N * D * E // 128, 128)
    return _tc_relayout(N, D * E)(x)


# speedup vs baseline: 1.3885x; 1.3885x over previous
import functools

import jax
import jax.numpy as jnp
from jax import lax
from jax.experimental import pallas as pl
from jax.experimental.pallas import tpu as pltpu
from jax.experimental.pallas import tpu_sc as plsc

_NC = 2
_NS = 16
_NW = _NC * _NS


def _build_gather(B: int, V: int, E: int, C: int, S: int):
    assert B % (_NW * C) == 0 and C % 8 == 0
    b_per_w = B // _NW
    nchunks = b_per_w // C
    assert nchunks % S == 0 and nchunks >= 2 * S

    mesh = plsc.VectorSubcoreMesh(core_axis_name="c", subcore_axis_name="s")

    @functools.partial(
        pl.kernel,
        out_type=jax.ShapeDtypeStruct((B, E), jnp.float32),
        mesh=mesh,
        compiler_params=pltpu.CompilerParams(use_tc_tiling_on_sc=False),
        scratch_types=(
            [pltpu.VMEM((C,), jnp.int32) for _ in range(S)]
            + [pltpu.VMEM((C, E), jnp.float32) for _ in range(S)]
            + [pltpu.SemaphoreType.DMA for _ in range(3 * S)]
        ),
    )
    def gather(idx_hbm, table_hbm, out_hbm, *refs):
        idx_v = refs[0:S]
        rows_v = refs[S:2 * S]
        sem_i = refs[2 * S:3 * S]
        sem_g = refs[3 * S:4 * S]
        sem_o = refs[4 * S:5 * S]
        wid = lax.axis_index("s") * _NC + lax.axis_index("c")
        base = wid * b_per_w

        def start_idx(g, b):
            pltpu.make_async_copy(
                idx_hbm.at[pl.ds(base + g * C, C)], idx_v[b], sem_i[b]
            ).start()

        def wait_idx(b):
            pltpu.make_async_copy(
                idx_hbm.at[pl.ds(base, C)], idx_v[b], sem_i[b]
            ).wait()

        def start_gather(b):
            pltpu.make_async_copy(
                table_hbm.at[idx_v[b]], rows_v[b], sem_g[b]
            ).start()

        def wait_gather(b):
            pltpu.make_async_copy(
                table_hbm.at[idx_v[b]], rows_v[b], sem_g[b]
            ).wait()

        def start_out(g, b):
            pltpu.make_async_copy(
                rows_v[b], out_hbm.at[pl.ds(base + g * C, C)], sem_o[b]
            ).start()

        def wait_out(b):
            pltpu.make_async_copy(
                rows_v[b], out_hbm.at[pl.ds(base, C)], sem_o[b]
            ).wait()

        def do_step(g, b, first=False, last=False):
            br = (b + 1) % S
            if not first:
                wait_out(b)
            wait_idx(b)
            start_gather(b)
            wait_gather(br)
            if not last:
                start_idx(g + 1, br)
            start_out(g - S + 1, br)

        for g in range(S):
            start_idx(g, g)
        for g in range(S - 1):
            wait_idx(g)
            start_gather(g)

        do_step(S - 1, S - 1, first=True)

        def loop_body(i, _):
            for o in range(S):
                do_step(i * S + o, o)
            return _

        lax.fori_loop(1, nchunks // S - 1, loop_body, 0, unroll=False)

        for g in range((nchunks // S - 1) * S, nchunks):
            do_step(g, g % S, last=(g == nchunks - 1))
        for q in range(nchunks - S + 1, nchunks):
            wait_gather(q % S)
            start_out(q, q % S)
        for b in range(S):
            wait_out(b)

    return gather


def kernel(p, table):
    N, D = p.shape
    V, E = table.shape
    flat = _build_gather(N * D, V, E, C=400, S=4)(p.reshape(N * D), table)
    return flat.reshape(N, D * E)


# speedup vs baseline: 2.0626x; 1.4855x over previous
import functools

import jax
import jax.numpy as jnp
from jax import lax
from jax.experimental import pallas as pl
from jax.experimental.pallas import tpu as pltpu
from jax.experimental.pallas import tpu_sc as plsc

_NC = 2
_NS = 16
_NW = _NC * _NS


def _build_gather(B: int, V: int, E: int, C: int, S: int):
    assert B % (_NW * C) == 0 and C % 8 == 0
    b_per_w = B // _NW
    nchunks = b_per_w // C
    assert nchunks % S == 0 and nchunks >= 2 * S

    mesh = plsc.VectorSubcoreMesh(core_axis_name="c", subcore_axis_name="s")

    @functools.partial(
        pl.kernel,
        out_type=jax.ShapeDtypeStruct((B, E), jnp.float32),
        mesh=mesh,
        compiler_params=pltpu.CompilerParams(use_tc_tiling_on_sc=False),
        scratch_types=(
            [pltpu.VMEM((C,), jnp.int32) for _ in range(S)]
            + [pltpu.VMEM((C, E), jnp.float32) for _ in range(S)]
            + [pltpu.SemaphoreType.DMA for _ in range(3 * S)]
            + [pltpu.VMEM_SHARED((V, E), jnp.float32)]
        ),
    )
    def gather(idx_hbm, table_hbm, out_hbm, *refs):
        idx_v = refs[0:S]
        rows_v = refs[S:2 * S]
        sem_i = refs[2 * S:3 * S]
        sem_g = refs[3 * S:4 * S]
        sem_o = refs[4 * S:5 * S]
        table_sh = refs[5 * S]
        sid = lax.axis_index("s")
        wid = sid * _NC + lax.axis_index("c")
        base = wid * b_per_w

        @pl.when(sid == 0)
        def _():
            pltpu.sync_copy(table_hbm, table_sh)

        plsc.subcore_barrier()

        def start_idx(g, b):
            pltpu.make_async_copy(
                idx_hbm.at[pl.ds(base + g * C, C)], idx_v[b], sem_i[b]
            ).start()

        def wait_idx(b):
            pltpu.make_async_copy(
                idx_hbm.at[pl.ds(base, C)], idx_v[b], sem_i[b]
            ).wait()

        def start_gather(b):
            pltpu.make_async_copy(
                table_sh.at[idx_v[b]], rows_v[b], sem_g[b]
            ).start()

        def wait_gather(b):
            pltpu.make_async_copy(
                table_sh.at[idx_v[b]], rows_v[b], sem_g[b]
            ).wait()

        def start_out(g, b):
            pltpu.make_async_copy(
                rows_v[b], out_hbm.at[pl.ds(base + g * C, C)], sem_o[b]
            ).start()

        def wait_out(b):
            pltpu.make_async_copy(
                rows_v[b], out_hbm.at[pl.ds(base, C)], sem_o[b]
            ).wait()

        def do_step(g, b, first=False, last=False):
            br = (b + 1) % S
            if not first:
                wait_out(b)
            wait_idx(b)
            start_gather(b)
            wait_gather(br)
            if not last:
                start_idx(g + 1, br)
            start_out(g - S + 1, br)

        for g in range(S):
            start_idx(g, g)
        for g in range(S - 1):
            wait_idx(g)
            start_gather(g)

        do_step(S - 1, S - 1, first=True)

        def loop_body(i, _):
            for o in range(S):
                do_step(i * S + o, o)
            return _

        lax.fori_loop(1, nchunks // S - 1, loop_body, 0, unroll=False)

        for g in range((nchunks // S - 1) * S, nchunks):
            do_step(g, g % S, last=(g == nchunks - 1))
        for q in range(nchunks - S + 1, nchunks):
            wait_gather(q % S)
            start_out(q, q % S)
        for b in range(S):
            wait_out(b)

    return gather


def kernel(p, table):
    N, D = p.shape
    V, E = table.shape
    flat = _build_gather(N * D, V, E, C=400, S=4)(p.reshape(N * D), table)
    return flat.reshape(N, D * E)
